# baseline (speedup 1.0000x reference)
import functools

import jax
import jax.numpy as jnp
from jax import lax
from jax.experimental import pallas as pl
from jax.experimental.pallas import tpu as pltpu
from jax.experimental.pallas import tpu_sc as plsc

N_CORES = 2
N_SUBCORES = 16
NW = N_CORES * N_SUBCORES
CH = 128


@functools.lru_cache(maxsize=None)
def _make_seg_sum(n_rows, n_pad, d, e_pad):
    per_w = e_pad // NW
    n_chunks = per_w // CH
    rows_per_tile = n_pad // N_SUBCORES
    n_row_chunks = rows_per_tile // CH
    mesh = plsc.VectorSubcoreMesh(core_axis_name="c", subcore_axis_name="s",
                                  num_cores=N_CORES, num_subcores=N_SUBCORES)

    @functools.partial(
        pl.kernel,
        out_type=jax.ShapeDtypeStruct((N_CORES, n_pad, d), jnp.float32),
        mesh=mesh,
        scratch_types=[
            pltpu.VMEM((CH,), jnp.int32),
            pltpu.VMEM((CH,), jnp.int32),
            pltpu.VMEM((CH, d), jnp.float32),
            pltpu.VMEM_SHARED((n_pad, d), jnp.float32),
            pltpu.SemaphoreType.DMA,
        ],
        compiler_params=pltpu.CompilerParams(use_tc_tiling_on_sc=False),
    )
    def seg_sum(vals_hbm, src_hbm, dst_hbm, zeros_hbm, out_hbm,
                sidx, didx, rows, acc, sem):
        cid = lax.axis_index("c")
        sid = lax.axis_index("s")
        wid = sid * N_CORES + cid
        r0 = sid * rows_per_tile
        for j in range(n_row_chunks):
            pltpu.sync_copy(zeros_hbm, acc.at[pl.ds(r0 + j * CH, CH)])
        plsc.subcore_barrier()

        def body(t, carry):
            base = (wid * n_chunks + t) * CH
            pltpu.sync_copy(src_hbm.at[pl.ds(base, CH)], sidx)
            pltpu.async_copy(vals_hbm.at[sidx], rows, sem).wait()
            pltpu.sync_copy(dst_hbm.at[pl.ds(base, CH)], didx)
            pltpu.sync_copy(rows, acc.at[didx], add=True)
            return carry

        lax.fori_loop(0, n_chunks, body, 0)
        plsc.subcore_barrier()
        for j in range(n_row_chunks):
            pltpu.sync_copy(acc.at[pl.ds(r0 + j * CH, CH)],
                            out_hbm.at[cid, pl.ds(r0 + j * CH, CH)])

    return seg_sum


def _lin_body(x_ref, w_ref, o_ref):
    o_ref[...] = jnp.dot(x_ref[...], w_ref[...],
                         preferred_element_type=jnp.float32)


def _layer0_body(p_ref, xw_ref, wl_ref, h0_ref, cnt_ref):
    s = p_ref[0] + p_ref[1]
    cnt = jnp.maximum(s[:, 7:8], 1.0)
    agg = s / cnt
    h0_ref[...] = jnp.tanh(
        jnp.dot(agg, wl_ref[...], preferred_element_type=jnp.float32)
        + xw_ref[...])
    cnt_ref[...] = cnt


def _layer1_body(p_ref, hw_ref, cnt_ref, wl_ref, b_ref, wz_ref, z_ref):
    agg = (p_ref[0] + p_ref[1]) / cnt_ref[...]
    h1 = jnp.tanh(
        jnp.dot(agg, wl_ref[...], preferred_element_type=jnp.float32)
        + hw_ref[...] + b_ref[...])
    z_ref[...] = jnp.dot(h1, wz_ref[...], preferred_element_type=jnp.float32)


def _heads_body(p_ref, z_ref, cnt_ref, xp_ref, sc_ref, actor_ref, critic_ref):
    n = z_ref.shape[0]
    aggz = (p_ref[0] + p_ref[1]) / cnt_ref[...]
    actor_pre = aggz[:, 0:1] + sc_ref[0, 0] + z_ref[:, 2:3]
    crit = aggz[:, 1:2] + sc_ref[0, 1] + z_ref[:, 3:4]
    mask = (xp_ref[:, 3:4] != 0.0) | (xp_ref[:, 4:5] != 0.0)
    a = jnp.where(mask, -jnp.inf, actor_pre)
    m = jnp.max(a)
    lse = m + jnp.log(jnp.sum(jnp.exp(a - m)))
    actor_ref[...] = a - lse
    critic_ref[...] = jnp.full((1, 1), jnp.tanh(jnp.sum(crit) / n), jnp.float32)


def kernel(x, edge_index, batch, Wl0, Wr0, b0, Wl1, Wr1, b1,
           Wla, Wra, ba, Wlc, Wrc, bc, Wf, bf):
    n = x.shape[0]
    e = edge_index.shape[1]
    n_pad = ((n + 2048) // 2048) * 2048
    blk = NW * CH
    e_pad = ((e + blk - 1) // blk) * blk

    src = edge_index[0]
    dst = edge_index[1]
    if e_pad != e:
        src = jnp.concatenate([src, jnp.zeros((e_pad - e,), jnp.int32)])
        dst = jnp.concatenate([dst, jnp.full((e_pad - e,), n, jnp.int32)])

    xpad = jnp.concatenate([x, jnp.ones((n, 1), jnp.float32)], axis=1)

    wl0p = jnp.concatenate([Wl0, jnp.zeros((1, Wl0.shape[1]), jnp.float32)])
    wr0p = jnp.concatenate([Wr0, b0[None, :]])
    wlcf = Wlc @ Wf
    wrcf = Wrc @ Wf
    wz = jnp.concatenate(
        [Wla, wlcf, Wra, wrcf, jnp.zeros((Wla.shape[0], 4), jnp.float32)],
        axis=1)
    scal = jnp.stack([ba[0], bc @ Wf[:, 0] + bf[0]])[None, :]

    z8 = jnp.zeros((CH, 8), jnp.float32)
    z128 = jnp.zeros((CH, 128), jnp.float32)

    grid = (n // 1000,)
    R = 1000

    def _lin(xv, w, dout):
        din = xv.shape[1]
        return pl.pallas_call(
            _lin_body,
            grid=grid,
            in_specs=[
                pl.BlockSpec((R, din), lambda i: (i, 0)),
                pl.BlockSpec((din, dout), lambda i: (0, 0)),
            ],
            out_specs=pl.BlockSpec((R, dout), lambda i: (i, 0)),
            out_shape=jax.ShapeDtypeStruct((n, dout), jnp.float32),
        )(xv, w)

    p8 = _make_seg_sum(n, n_pad, 8, e_pad)(xpad, src, dst, z8)
    xw = _lin(xpad, wr0p, 128)

    h0, cnt = pl.pallas_call(
        _layer0_body,
        grid=grid,
        in_specs=[
            pl.BlockSpec((N_CORES, R, 8), lambda i: (0, i, 0)),
            pl.BlockSpec((R, 128), lambda i: (i, 0)),
            pl.BlockSpec((8, 128), lambda i: (0, 0)),
        ],
        out_specs=[
            pl.BlockSpec((R, 128), lambda i: (i, 0)),
            pl.BlockSpec((R, 1), lambda i: (i, 0)),
        ],
        out_shape=[
            jax.ShapeDtypeStruct((n, 128), jnp.float32),
            jax.ShapeDtypeStruct((n, 1), jnp.float32),
        ],
    )(p8, xw, wl0p)

    p128 = _make_seg_sum(n, n_pad, 128, e_pad)(h0, src, dst, z128)
    hw = _lin(h0, Wr1, 128)

    z = pl.pallas_call(
        _layer1_body,
        grid=grid,
        in_specs=[
            pl.BlockSpec((N_CORES, R, 128), lambda i: (0, i, 0)),
            pl.BlockSpec((R, 128), lambda i: (i, 0)),
            pl.BlockSpec((R, 1), lambda i: (i, 0)),
            pl.BlockSpec((128, 128), lambda i: (0, 0)),
            pl.BlockSpec((1, 128), lambda i: (0, 0)),
            pl.BlockSpec((128, 8), lambda i: (0, 0)),
        ],
        out_specs=pl.BlockSpec((R, 8), lambda i: (i, 0)),
        out_shape=jax.ShapeDtypeStruct((n, 8), jnp.float32),
    )(p128, hw, cnt, Wl1, b1[None, :], wz)

    pz = _make_seg_sum(n, n_pad, 8, e_pad)(z, src, dst, z8)

    x_actor, x_critic = pl.pallas_call(
        _heads_body,
        grid=(1,),
        in_specs=[
            pl.BlockSpec((N_CORES, n, 8), lambda i: (0, 0, 0)),
            pl.BlockSpec((n, 8), lambda i: (0, 0)),
            pl.BlockSpec((n, 1), lambda i: (0, 0)),
            pl.BlockSpec((n, 8), lambda i: (0, 0)),
            pl.BlockSpec((1, 2), lambda i: (0, 0)),
        ],
        out_specs=[
            pl.BlockSpec((n, 1), lambda i: (0, 0)),
            pl.BlockSpec((1, 1), lambda i: (0, 0)),
        ],
        out_shape=[
            jax.ShapeDtypeStruct((n, 1), jnp.float32),
            jax.ShapeDtypeStruct((1, 1), jnp.float32),
        ],
    )(pz, z, cnt, xpad, scal)

    return (x_actor, x_critic)


# speedup vs baseline: 1.0668x; 1.0668x over previous
import functools

import jax
import jax.numpy as jnp
from jax import lax
from jax.experimental import pallas as pl
from jax.experimental.pallas import tpu as pltpu
from jax.experimental.pallas import tpu_sc as plsc

N_CORES = 2
N_SUBCORES = 16
NW = N_CORES * N_SUBCORES
CH = 128


@functools.lru_cache(maxsize=None)
def _make_seg_sum(n_rows, n_pad, d, e_pad):
    per_w = e_pad // NW
    n_chunks = per_w // CH
    rows_per_tile = n_pad // N_SUBCORES
    n_row_chunks = rows_per_tile // CH
    mesh = plsc.VectorSubcoreMesh(core_axis_name="c", subcore_axis_name="s",
                                  num_cores=N_CORES, num_subcores=N_SUBCORES)

    @functools.partial(
        pl.kernel,
        out_type=jax.ShapeDtypeStruct((N_CORES, n_pad, d), jnp.float32),
        mesh=mesh,
        scratch_types=[
            pltpu.VMEM((CH,), jnp.int32),
            pltpu.VMEM((CH,), jnp.int32),
            pltpu.VMEM((CH, d), jnp.float32),
            pltpu.VMEM_SHARED((n_pad, d), jnp.float32),
            pltpu.SemaphoreType.DMA,
        ],
        compiler_params=pltpu.CompilerParams(use_tc_tiling_on_sc=False),
    )
    def seg_sum(vals_hbm, src_hbm, dst_hbm, zeros_hbm, out_hbm,
                sidx, didx, rows, acc, sem):
        cid = lax.axis_index("c")
        sid = lax.axis_index("s")
        wid = sid * N_CORES + cid
        r0 = sid * rows_per_tile
        for j in range(n_row_chunks):
            pltpu.sync_copy(zeros_hbm, acc.at[pl.ds(r0 + j * CH, CH)])
        plsc.subcore_barrier()

        def body(t, carry):
            base = (wid * n_chunks + t) * CH
            pltpu.sync_copy(src_hbm.at[pl.ds(base, CH)], sidx)
            pltpu.async_copy(vals_hbm.at[sidx], rows, sem).wait()
            pltpu.sync_copy(dst_hbm.at[pl.ds(base, CH)], didx)
            pltpu.sync_copy(rows, acc.at[didx], add=True)
            return carry

        lax.fori_loop(0, n_chunks, body, 0)
        plsc.subcore_barrier()
        for j in range(n_row_chunks):
            pltpu.sync_copy(acc.at[pl.ds(r0 + j * CH, CH)],
                            out_hbm.at[cid, pl.ds(r0 + j * CH, CH)])

    return seg_sum


def _layer0_body(p_ref, xp_ref, wl_ref, wr_ref, h0_ref, cnt_ref):
    s = p_ref[0] + p_ref[1]
    cnt = jnp.maximum(s[:, 7:8], 1.0)
    agg = s / cnt
    h0_ref[...] = jnp.tanh(
        jnp.dot(agg, wl_ref[...], preferred_element_type=jnp.float32)
        + jnp.dot(xp_ref[...], wr_ref[...], preferred_element_type=jnp.float32))
    cnt_ref[...] = cnt


def _layer1_body(p_ref, h0_ref, cnt_ref, wl_ref, wr_ref, b_ref, wz_ref, z_ref):
    agg = (p_ref[0] + p_ref[1]) / cnt_ref[...]
    h1 = jnp.tanh(
        jnp.dot(agg, wl_ref[...], preferred_element_type=jnp.float32)
        + jnp.dot(h0_ref[...], wr_ref[...], preferred_element_type=jnp.float32)
        + b_ref[...])
    z_ref[...] = jnp.dot(h1, wz_ref[...], preferred_element_type=jnp.float32)


def _heads_body(p_ref, z_ref, cnt_ref, xp_ref, sc_ref, actor_ref, critic_ref):
    n = z_ref.shape[0]
    aggz = (p_ref[0] + p_ref[1]) / cnt_ref[...]
    actor_pre = aggz[:, 0:1] + sc_ref[0, 0] + z_ref[:, 2:3]
    crit = aggz[:, 1:2] + sc_ref[0, 1] + z_ref[:, 3:4]
    mask = (xp_ref[:, 3:4] != 0.0) | (xp_ref[:, 4:5] != 0.0)
    a = jnp.where(mask, -jnp.inf, actor_pre)
    m = jnp.max(a)
    lse = m + jnp.log(jnp.sum(jnp.exp(a - m)))
    actor_ref[...] = a - lse
    critic_ref[...] = jnp.full((1, 1), jnp.tanh(jnp.sum(crit) / n), jnp.float32)


def kernel(x, edge_index, batch, Wl0, Wr0, b0, Wl1, Wr1, b1,
           Wla, Wra, ba, Wlc, Wrc, bc, Wf, bf):
    n = x.shape[0]
    e = edge_index.shape[1]
    n_pad = ((n + 2048) // 2048) * 2048
    blk = NW * CH
    e_pad = ((e + blk - 1) // blk) * blk

    src = edge_index[0]
    dst = edge_index[1]
    if e_pad != e:
        src = jnp.concatenate([src, jnp.zeros((e_pad - e,), jnp.int32)])
        dst = jnp.concatenate([dst, jnp.full((e_pad - e,), n, jnp.int32)])

    xpad = jnp.concatenate([x, jnp.ones((n, 1), jnp.float32)], axis=1)

    wl0p = jnp.concatenate([Wl0, jnp.zeros((1, Wl0.shape[1]), jnp.float32)])
    wr0p = jnp.concatenate([Wr0, b0[None, :]])
    wlcf = Wlc @ Wf
    wrcf = Wrc @ Wf
    wz = jnp.concatenate(
        [Wla, wlcf, Wra, wrcf, jnp.zeros((Wla.shape[0], 4), jnp.float32)],
        axis=1)
    scal = jnp.stack([ba[0], bc @ Wf[:, 0] + bf[0]])[None, :]

    z8 = jnp.zeros((CH, 8), jnp.float32)
    z128 = jnp.zeros((CH, 128), jnp.float32)

    p8 = _make_seg_sum(n, n_pad, 8, e_pad)(xpad, src, dst, z8)

    grid = (n // 1000,)
    R = 1000
    h0, cnt = pl.pallas_call(
        _layer0_body,
        grid=grid,
        in_specs=[
            pl.BlockSpec((N_CORES, R, 8), lambda i: (0, i, 0)),
            pl.BlockSpec((R, 8), lambda i: (i, 0)),
            pl.BlockSpec((8, 128), lambda i: (0, 0)),
            pl.BlockSpec((8, 128), lambda i: (0, 0)),
        ],
        out_specs=[
            pl.BlockSpec((R, 128), lambda i: (i, 0)),
            pl.BlockSpec((R, 1), lambda i: (i, 0)),
        ],
        out_shape=[
            jax.ShapeDtypeStruct((n, 128), jnp.float32),
            jax.ShapeDtypeStruct((n, 1), jnp.float32),
        ],
    )(p8, xpad, wl0p, wr0p)

    p128 = _make_seg_sum(n, n_pad, 128, e_pad)(h0, src, dst, z128)

    z = pl.pallas_call(
        _layer1_body,
        grid=grid,
        in_specs=[
            pl.BlockSpec((N_CORES, R, 128), lambda i: (0, i, 0)),
            pl.BlockSpec((R, 128), lambda i: (i, 0)),
            pl.BlockSpec((R, 1), lambda i: (i, 0)),
            pl.BlockSpec((128, 128), lambda i: (0, 0)),
            pl.BlockSpec((128, 128), lambda i: (0, 0)),
            pl.BlockSpec((1, 128), lambda i: (0, 0)),
            pl.BlockSpec((128, 8), lambda i: (0, 0)),
        ],
        out_specs=pl.BlockSpec((R, 8), lambda i: (i, 0)),
        out_shape=jax.ShapeDtypeStruct((n, 8), jnp.float32),
    )(p128, h0, cnt, Wl1, Wr1, b1[None, :], wz)

    pz = _make_seg_sum(n, n_pad, 8, e_pad)(z, src, dst, z8)

    x_actor, x_critic = pl.pallas_call(
        _heads_body,
        grid=(1,),
        in_specs=[
            pl.BlockSpec((N_CORES, n, 8), lambda i: (0, 0, 0)),
            pl.BlockSpec((n, 8), lambda i: (0, 0)),
            pl.BlockSpec((n, 1), lambda i: (0, 0)),
            pl.BlockSpec((n, 8), lambda i: (0, 0)),
            pl.BlockSpec((1, 2), lambda i: (0, 0)),
        ],
        out_specs=[
            pl.BlockSpec((n, 1), lambda i: (0, 0)),
            pl.BlockSpec((1, 1), lambda i: (0, 0)),
        ],
        out_shape=[
            jax.ShapeDtypeStruct((n, 1), jnp.float32),
            jax.ShapeDtypeStruct((1, 1), jnp.float32),
        ],
    )(pz, z, cnt, xpad, scal)

    return (x_actor, x_critic)
